# baseline (speedup 1.0000x reference)
import functools

import jax
import jax.numpy as jnp
from jax import lax
from jax.experimental import pallas as pl
from jax.experimental.pallas import tpu as pltpu
from jax.experimental.pallas import tpu_sc as plsc

N = 10000
D = 256
E = 160000
H = D // 2

NS = 16
L = 16

NPAD = 10240
RPT = NPAD // NS

EPT = E // NS
CH = 40
NBUF = 5
NCH = EPT // CH
EROWS = E // CH
ACH = 80
ANCH = EPT // ACH
AROWS = E // ACH
TROWS = N // NS
ZCH = 25
NZ = TROWS // ZCH

def _mesh():
    return plsc.VectorSubcoreMesh(core_axis_name="c", subcore_axis_name="s",
                                  num_cores=2, num_subcores=NS)


def _zero_1d(ref, n):
    def body(i, _):
        ref[pl.ds(i * L, L)] = jnp.zeros((L,), jnp.float32)
        return 0

    lax.fori_loop(0, n // L, body, 0)


def _zero_2d(ref, rows, cols):
    def body(i, _):
        for k in range(cols // L):
            ref[i, pl.ds(k * L, L)] = jnp.zeros((L,), jnp.float32)
        return 0

    lax.fori_loop(0, rows, body, 0)


@functools.cache
def _degrees_kernel():
    return pl.kernel(
        _degrees_body,
        out_type=[jax.ShapeDtypeStruct((NPAD,), jnp.float32) for _ in range(4)],
        mesh=_mesh(),
        scratch_types=[
            pltpu.VMEM((NPAD,), jnp.float32),
            pltpu.VMEM((NPAD,), jnp.float32),
            pltpu.VMEM((ANCH, ACH), jnp.int32),
            pltpu.VMEM_SHARED((NS, 2, NPAD), jnp.float32),
            pltpu.VMEM((NS, RPT), jnp.float32),
            pltpu.VMEM((RPT,), jnp.float32),
        ],
        compiler_params=pltpu.CompilerParams(use_tc_tiling_on_sc=False, needs_layout_passes=False),
    )


def _degrees_body(src_k, dst_k, src_l, dst_l,
                  deg_ok, deg_ik, deg_ol, deg_il,
                  hist_o, hist_i, idx_v, partials, red2d_v, red_v):
    c = lax.axis_index("c")
    s = lax.axis_index("s")
    ones = jnp.ones((L,), jnp.float32)

    def count(src_hbm, hist):
        pltpu.sync_copy(src_hbm.at[pl.ds(s * ANCH, ANCH)], idx_v)

        def body(j, _):
            for k in range(ACH // L):
                idx = idx_v[j, pl.ds(k * L, L)]
                plsc.addupdate_scatter(hist, [idx], ones)
            return 0

        lax.fori_loop(0, ANCH, body, 0)

    def reduce_out(a, out_hbm):
        pltpu.sync_copy(partials.at[:, a, pl.ds(s * RPT, RPT)], red2d_v)

        def body_j(j, _):
            sl = pl.ds(j * L, L)
            tot = red2d_v[0, sl]
            for k in range(1, NS):
                tot = tot + red2d_v[k, sl]
            red_v[sl] = tot
            return 0

        lax.fori_loop(0, RPT // L, body_j, 0)
        pltpu.sync_copy(red_v, out_hbm.at[pl.ds(s * RPT, RPT)])

    def run(src_hbm, dst_hbm, out_o, out_i):
        _zero_1d(hist_o, NPAD)
        _zero_1d(hist_i, NPAD)
        count(src_hbm, hist_o)
        count(dst_hbm, hist_i)
        pltpu.sync_copy(hist_o, partials.at[s, 0])
        pltpu.sync_copy(hist_i, partials.at[s, 1])
        plsc.subcore_barrier()
        reduce_out(0, out_o)
        reduce_out(1, out_i)

    @pl.when(c == 0)
    def _():
        run(src_k, dst_k, deg_ok, deg_ik)

    @pl.when(c == 1)
    def _():
        run(src_l, dst_l, deg_ol, deg_il)


def _mm_body(x_ref, deg_ref, w_ref, out0_ref, out1_ref):
    d = deg_ref[...]
    nrm = jnp.where(d > 0, lax.rsqrt(jnp.maximum(d, 1e-12)), 0.0)
    xb = x_ref[...] * nrm
    y = jnp.dot(xb, w_ref[...], preferred_element_type=jnp.float32)
    out0_ref[...] = y[:, :H]
    out1_ref[...] = y[:, H:]


def _premultiplied_matmul(x, deg, w):
    bn = 1000
    row = lambda i: (i, 0)
    return pl.pallas_call(
        _mm_body,
        grid=(N // bn,),
        in_specs=[
            pl.BlockSpec((bn, D), row),
            pl.BlockSpec((bn, 1), row),
            pl.BlockSpec((D, D), lambda i: (0, 0)),
        ],
        out_specs=[pl.BlockSpec((bn, H), row), pl.BlockSpec((bn, H), row)],
        out_shape=[jax.ShapeDtypeStruct((N, H), jnp.float32) for _ in range(2)],
    )(x, deg, w)


@functools.cache
def _scatter_kernel():
    return pl.kernel(
        _scatter_body,
        out_type=[jax.ShapeDtypeStruct((N, H), jnp.float32) for _ in range(4)],
        mesh=_mesh(),
        scratch_types=[
            pltpu.VMEM_SHARED((N, H), jnp.float32),
            pltpu.VMEM((NCH, CH), jnp.int32),
            pltpu.VMEM((NCH, CH), jnp.int32),
            *[pltpu.VMEM((CH, H), jnp.float32) for _ in range(NBUF)],
            pltpu.VMEM((ZCH, H), jnp.float32),
            *[pltpu.SemaphoreType.DMA for _ in range(2 * NBUF)],
        ],
        compiler_params=pltpu.CompilerParams(use_tc_tiling_on_sc=False, needs_layout_passes=False),
    )


def _scatter_body(yk0_hbm, yk1_hbm, yl0_hbm, yl1_hbm,
                  srck_hbm, dstk_hbm, srcl_hbm, dstl_hbm,
                  outk0, outk1, outl0, outl1,
                  acc, src_v, dst_v, r0, r1, r2, r3, r4, stage_v,
                  g0, g1, g2, g3, g4, s0, s1, s2, s3, s4):
    bufs = (r0, r1, r2, r3, r4)
    gsems = (g0, g1, g2, g3, g4)
    ssems = (s0, s1, s2, s3, s4)
    c = lax.axis_index("c")
    s = lax.axis_index("s")

    def zero_acc():
        def zbody(z, _):
            pltpu.sync_copy(stage_v, acc.at[pl.ds(s * TROWS + z * ZCH, ZCH)])
            return 0

        lax.fori_loop(0, NZ, zbody, 0)

    def run(y_half):
        def gather(j, t):
            pltpu.async_copy(y_half.at[src_v.at[j]], bufs[t], gsems[t])

        def wait_gather(t):
            pltpu.make_async_copy(y_half.at[src_v.at[0]], bufs[t],
                                  gsems[t]).wait()

        def scatter(j, t):
            pltpu.async_copy(bufs[t], acc.at[dst_v.at[j]], ssems[t],
                             add=True)

        def wait_scatter(t):
            pltpu.make_async_copy(bufs[t], acc.at[dst_v.at[0]],
                                  ssems[t]).wait()

        for t in range(NBUF):
            gather(t, t)

        ngroups = NCH // NBUF
        tail = NCH % NBUF

        def body(g, _):
            j0 = NBUF * g
            for t in range(NBUF):
                wait_gather(t)
                scatter(j0 + t, t)
            for t in range(NBUF):
                j2 = j0 + NBUF + t
                wait_scatter(t)

                @pl.when(j2 < NCH)
                def _():
                    gather(j2, t)

            return 0

        lax.fori_loop(0, ngroups, body, 0)
        for t in range(tail):
            wait_gather(t)
            scatter(NCH - tail + t, t)
            wait_scatter(t)

    def writeout(out0, out1):
        def wbody(z, _):
            rows = pl.ds(s * TROWS + z * ZCH, ZCH)
            pltpu.sync_copy(acc.at[rows], stage_v)

            @pl.when(c == 0)
            def _():
                pltpu.sync_copy(stage_v, out0.at[rows])

            @pl.when(c == 1)
            def _():
                pltpu.sync_copy(stage_v, out1.at[rows])

            return 0

        lax.fori_loop(0, NZ, wbody, 0)

    def relation(src_hbm, dst_hbm, y0_hbm, y1_hbm, out0, out1, zero_v):
        _zero_2d(zero_v, ZCH, H)
        zero_acc()
        pltpu.sync_copy(src_hbm.at[pl.ds(s * NCH, NCH)], src_v)
        pltpu.sync_copy(dst_hbm.at[pl.ds(s * NCH, NCH)], dst_v)
        plsc.subcore_barrier()

        @pl.when(c == 0)
        def _():
            run(y0_hbm)

        @pl.when(c == 1)
        def _():
            run(y1_hbm)

        plsc.subcore_barrier()
        writeout(out0, out1)

    relation(srck_hbm, dstk_hbm, yk0_hbm, yk1_hbm, outk0, outk1, stage_v)
    relation(srcl_hbm, dstl_hbm, yl0_hbm, yl1_hbm, outl0, outl1, stage_v)


def _epilogue_body(ak0, ak1, al0, al1, dk, dl, bk, bl, g, b, out_ref):
    def half(a0, a1, dref, bref):
        a = jnp.concatenate([a0[...], a1[...]], axis=1)
        d = dref[...]
        nrm = jnp.where(d > 0, lax.rsqrt(jnp.maximum(d, 1e-12)), 0.0)
        return jax.nn.relu(a * nrm + bref[...])

    hk = half(ak0, ak1, dk, bk)
    hl = half(al0, al1, dl, bl)
    m = jnp.maximum(hk, hl)
    lse = m + jnp.log1p(jnp.exp(-jnp.abs(hk - hl)))
    mu = jnp.mean(lse, axis=1, keepdims=True)
    cent = lse - mu
    var = jnp.mean(cent * cent, axis=1, keepdims=True)
    out_ref[...] = cent * lax.rsqrt(var + 1e-6) * g[...] + b[...]


def _epilogue(agg_k0, agg_k1, agg_l0, agg_l1, deg_ik, deg_il,
              b_knows, b_likes, gamma, beta):
    bn = 1000
    row = lambda i: (i, 0)
    vec = lambda i: (0,)
    return pl.pallas_call(
        _epilogue_body,
        grid=(N // bn,),
        in_specs=[
            pl.BlockSpec((bn, H), row),
            pl.BlockSpec((bn, H), row),
            pl.BlockSpec((bn, H), row),
            pl.BlockSpec((bn, H), row),
            pl.BlockSpec((bn, 1), row),
            pl.BlockSpec((bn, 1), row),
            pl.BlockSpec((D,), vec),
            pl.BlockSpec((D,), vec),
            pl.BlockSpec((D,), vec),
            pl.BlockSpec((D,), vec),
        ],
        out_specs=pl.BlockSpec((bn, D), row),
        out_shape=jax.ShapeDtypeStruct((N, D), jnp.float32),
    )(agg_k0, agg_k1, agg_l0, agg_l1, deg_ik, deg_il,
      b_knows, b_likes, gamma, beta)


def kernel(x_knows, x_likes, edge_index_knows, edge_index_likes,
           W_knows, b_knows, W_likes, b_likes, gamma, beta):
    src_k = edge_index_knows[0].astype(jnp.int32)
    dst_k = edge_index_knows[1].astype(jnp.int32)
    src_l = edge_index_likes[0].astype(jnp.int32)
    dst_l = edge_index_likes[1].astype(jnp.int32)

    deg_ok, deg_ik, deg_ol, deg_il = _degrees_kernel()(
        src_k.reshape(AROWS, ACH), dst_k.reshape(AROWS, ACH),
        src_l.reshape(AROWS, ACH), dst_l.reshape(AROWS, ACH))
    deg_ok = deg_ok[:N].reshape(N, 1)
    deg_ik = deg_ik[:N].reshape(N, 1)
    deg_ol = deg_ol[:N].reshape(N, 1)
    deg_il = deg_il[:N].reshape(N, 1)

    y_k0, y_k1 = _premultiplied_matmul(x_knows, deg_ok, W_knows)
    y_l0, y_l1 = _premultiplied_matmul(x_likes, deg_ol, W_likes)

    agg_k0, agg_k1, agg_l0, agg_l1 = _scatter_kernel()(
        y_k0, y_k1, y_l0, y_l1,
        src_k.reshape(EROWS, CH), dst_k.reshape(EROWS, CH),
        src_l.reshape(EROWS, CH), dst_l.reshape(EROWS, CH))

    return _epilogue(agg_k0, agg_k1, agg_l0, agg_l1, deg_ik, deg_il,
                     b_knows, b_likes, gamma, beta)


# speedup vs baseline: 1.0526x; 1.0526x over previous
import functools

import jax
import jax.numpy as jnp
from jax import lax
from jax.experimental import pallas as pl
from jax.experimental.pallas import tpu as pltpu
from jax.experimental.pallas import tpu_sc as plsc

N = 10000
D = 256
E = 160000
H = D // 2

NS = 16
L = 16

NPAD = 10240
RPT = NPAD // NS

EPT = E // NS
CH = 40
NBUF = 5
NCH = EPT // CH
EROWS = E // CH
ACH = 80
ANCH = EPT // ACH
AROWS = E // ACH
TROWS = N // NS
ZCH = 25
NZ = TROWS // ZCH

def _mesh():
    return plsc.VectorSubcoreMesh(core_axis_name="c", subcore_axis_name="s",
                                  num_cores=2, num_subcores=NS)


def _zero_1d(ref, n):
    def body(i, _):
        ref[pl.ds(i * L, L)] = jnp.zeros((L,), jnp.float32)
        return 0

    lax.fori_loop(0, n // L, body, 0)


def _zero_2d(ref, rows, cols):
    def body(i, _):
        for k in range(cols // L):
            ref[i, pl.ds(k * L, L)] = jnp.zeros((L,), jnp.float32)
        return 0

    lax.fori_loop(0, rows, body, 0)


@functools.cache
def _degrees_kernel():
    return pl.kernel(
        _degrees_body,
        out_type=[jax.ShapeDtypeStruct((NPAD,), jnp.float32) for _ in range(4)],
        mesh=_mesh(),
        scratch_types=[
            pltpu.VMEM((NPAD,), jnp.float32),
            pltpu.VMEM((NPAD,), jnp.float32),
            pltpu.VMEM((ANCH, ACH), jnp.int32),
            pltpu.VMEM_SHARED((NS, 2, NPAD), jnp.float32),
            pltpu.VMEM((NS, RPT), jnp.float32),
            pltpu.VMEM((RPT,), jnp.float32),
        ],
        compiler_params=pltpu.CompilerParams(use_tc_tiling_on_sc=False, needs_layout_passes=False),
    )


def _degrees_body(src_k, dst_k, src_l, dst_l,
                  deg_ok, deg_ik, deg_ol, deg_il,
                  hist_o, hist_i, idx_v, partials, red2d_v, red_v):
    c = lax.axis_index("c")
    s = lax.axis_index("s")
    ones = jnp.ones((L,), jnp.float32)

    def count(src_hbm, hist):
        pltpu.sync_copy(src_hbm.at[pl.ds(s * ANCH, ANCH)], idx_v)

        def body(j, _):
            for k in range(ACH // L):
                idx = idx_v[j, pl.ds(k * L, L)]
                plsc.addupdate_scatter(hist, [idx], ones)
            return 0

        lax.fori_loop(0, ANCH, body, 0)

    def reduce_out(a, out_hbm):
        pltpu.sync_copy(partials.at[:, a, pl.ds(s * RPT, RPT)], red2d_v)

        def body_j(j, _):
            sl = pl.ds(j * L, L)
            tot = red2d_v[0, sl]
            for k in range(1, NS):
                tot = tot + red2d_v[k, sl]
            red_v[sl] = tot
            return 0

        lax.fori_loop(0, RPT // L, body_j, 0)
        pltpu.sync_copy(red_v, out_hbm.at[pl.ds(s * RPT, RPT)])

    def run(src_hbm, dst_hbm, out_o, out_i):
        _zero_1d(hist_o, NPAD)
        _zero_1d(hist_i, NPAD)
        count(src_hbm, hist_o)
        count(dst_hbm, hist_i)
        pltpu.sync_copy(hist_o, partials.at[s, 0])
        pltpu.sync_copy(hist_i, partials.at[s, 1])
        plsc.subcore_barrier()
        reduce_out(0, out_o)
        reduce_out(1, out_i)

    @pl.when(c == 0)
    def _():
        run(src_k, dst_k, deg_ok, deg_ik)

    @pl.when(c == 1)
    def _():
        run(src_l, dst_l, deg_ol, deg_il)


def _mm_body(x_ref, deg_ref, w_ref, out0_ref, out1_ref):
    d = deg_ref[...]
    nrm = jnp.where(d > 0, lax.rsqrt(jnp.maximum(d, 1e-12)), 0.0)
    xb = x_ref[...] * nrm
    y = jnp.dot(xb, w_ref[...], preferred_element_type=jnp.float32)
    out0_ref[...] = y[:, :H]
    out1_ref[...] = y[:, H:]


def _premultiplied_matmul(x, deg, w):
    bn = 1000
    row = lambda i: (i, 0)
    return pl.pallas_call(
        _mm_body,
        grid=(N // bn,),
        in_specs=[
            pl.BlockSpec((bn, D), row),
            pl.BlockSpec((bn, 1), row),
            pl.BlockSpec((D, D), lambda i: (0, 0)),
        ],
        out_specs=[pl.BlockSpec((bn, H), row), pl.BlockSpec((bn, H), row)],
        out_shape=[jax.ShapeDtypeStruct((N, H), jnp.float32) for _ in range(2)],
    )(x, deg, w)


@functools.cache
def _scatter_kernel():
    return pl.kernel(
        _scatter_body,
        out_type=[jax.ShapeDtypeStruct((N, H), jnp.float32) for _ in range(2)],
        mesh=_mesh(),
        scratch_types=[
            pltpu.VMEM_SHARED((N, H), jnp.float32),
            pltpu.VMEM((NCH, CH), jnp.int32),
            pltpu.VMEM((NCH, CH), jnp.int32),
            *[pltpu.VMEM((CH, H), jnp.float32) for _ in range(NBUF)],
            pltpu.VMEM((ZCH, H), jnp.float32),
            *[pltpu.SemaphoreType.DMA for _ in range(2 * NBUF)],
        ],
        compiler_params=pltpu.CompilerParams(use_tc_tiling_on_sc=False, needs_layout_passes=False),
    )


def _scatter_body(y0_hbm, y1_hbm, src_hbm, dst_hbm,
                  out0, out1,
                  acc, src_v, dst_v, r0, r1, r2, r3, r4, stage_v,
                  g0, g1, g2, g3, g4, s0, s1, s2, s3, s4):
    bufs = (r0, r1, r2, r3, r4)
    gsems = (g0, g1, g2, g3, g4)
    ssems = (s0, s1, s2, s3, s4)
    c = lax.axis_index("c")
    s = lax.axis_index("s")

    def zero_acc():
        def zbody(z, _):
            pltpu.sync_copy(stage_v, acc.at[pl.ds(s * TROWS + z * ZCH, ZCH)])
            return 0

        lax.fori_loop(0, NZ, zbody, 0)

    def run(y_half):
        def gather(j, t):
            pltpu.async_copy(y_half.at[src_v.at[j]], bufs[t], gsems[t])

        def wait_gather(t):
            pltpu.make_async_copy(y_half.at[src_v.at[0]], bufs[t],
                                  gsems[t]).wait()

        def scatter(j, t):
            pltpu.async_copy(bufs[t], acc.at[dst_v.at[j]], ssems[t],
                             add=True)

        def wait_scatter(t):
            pltpu.make_async_copy(bufs[t], acc.at[dst_v.at[0]],
                                  ssems[t]).wait()

        for t in range(NBUF):
            gather(t, t)

        ngroups = NCH // NBUF
        tail = NCH % NBUF

        def body(g, _):
            j0 = NBUF * g
            for t in range(NBUF):
                wait_gather(t)
                scatter(j0 + t, t)
            for t in range(NBUF):
                j2 = j0 + NBUF + t
                wait_scatter(t)

                @pl.when(j2 < NCH)
                def _():
                    gather(j2, t)

            return 0

        lax.fori_loop(0, ngroups, body, 0)
        for t in range(tail):
            wait_gather(t)
            scatter(NCH - tail + t, t)
            wait_scatter(t)

    def writeout(out0, out1):
        rows = pl.ds(s * TROWS, TROWS)

        @pl.when(c == 0)
        def _():
            pltpu.sync_copy(acc.at[rows], out0.at[rows])

        @pl.when(c == 1)
        def _():
            pltpu.sync_copy(acc.at[rows], out1.at[rows])

    _zero_2d(stage_v, ZCH, H)
    zero_acc()
    pltpu.sync_copy(src_hbm.at[pl.ds(s * NCH, NCH)], src_v)
    pltpu.sync_copy(dst_hbm.at[pl.ds(s * NCH, NCH)], dst_v)
    plsc.subcore_barrier()

    @pl.when(c == 0)
    def _():
        run(y0_hbm)

    @pl.when(c == 1)
    def _():
        run(y1_hbm)

    plsc.subcore_barrier()
    writeout(out0, out1)


def _epilogue_body(ak0, ak1, al0, al1, dk, dl, bk, bl, g, b, out_ref):
    def half(a0, a1, dref, bref):
        a = jnp.concatenate([a0[...], a1[...]], axis=1)
        d = dref[...]
        nrm = jnp.where(d > 0, lax.rsqrt(jnp.maximum(d, 1e-12)), 0.0)
        return jax.nn.relu(a * nrm + bref[...])

    hk = half(ak0, ak1, dk, bk)
    hl = half(al0, al1, dl, bl)
    m = jnp.maximum(hk, hl)
    lse = m + jnp.log1p(jnp.exp(-jnp.abs(hk - hl)))
    mu = jnp.mean(lse, axis=1, keepdims=True)
    cent = lse - mu
    var = jnp.mean(cent * cent, axis=1, keepdims=True)
    out_ref[...] = cent * lax.rsqrt(var + 1e-6) * g[...] + b[...]


def _epilogue(agg_k0, agg_k1, agg_l0, agg_l1, deg_ik, deg_il,
              b_knows, b_likes, gamma, beta):
    bn = 1000
    row = lambda i: (i, 0)
    vec = lambda i: (0,)
    return pl.pallas_call(
        _epilogue_body,
        grid=(N // bn,),
        in_specs=[
            pl.BlockSpec((bn, H), row),
            pl.BlockSpec((bn, H), row),
            pl.BlockSpec((bn, H), row),
            pl.BlockSpec((bn, H), row),
            pl.BlockSpec((bn, 1), row),
            pl.BlockSpec((bn, 1), row),
            pl.BlockSpec((D,), vec),
            pl.BlockSpec((D,), vec),
            pl.BlockSpec((D,), vec),
            pl.BlockSpec((D,), vec),
        ],
        out_specs=pl.BlockSpec((bn, D), row),
        out_shape=jax.ShapeDtypeStruct((N, D), jnp.float32),
    )(agg_k0, agg_k1, agg_l0, agg_l1, deg_ik, deg_il,
      b_knows, b_likes, gamma, beta)


def kernel(x_knows, x_likes, edge_index_knows, edge_index_likes,
           W_knows, b_knows, W_likes, b_likes, gamma, beta):
    src_k = edge_index_knows[0].astype(jnp.int32)
    dst_k = edge_index_knows[1].astype(jnp.int32)
    src_l = edge_index_likes[0].astype(jnp.int32)
    dst_l = edge_index_likes[1].astype(jnp.int32)

    deg_ok, deg_ik, deg_ol, deg_il = _degrees_kernel()(
        src_k.reshape(AROWS, ACH), dst_k.reshape(AROWS, ACH),
        src_l.reshape(AROWS, ACH), dst_l.reshape(AROWS, ACH))
    deg_ok = deg_ok[:N].reshape(N, 1)
    deg_ik = deg_ik[:N].reshape(N, 1)
    deg_ol = deg_ol[:N].reshape(N, 1)
    deg_il = deg_il[:N].reshape(N, 1)

    y_k0, y_k1 = _premultiplied_matmul(x_knows, deg_ok, W_knows)
    y_l0, y_l1 = _premultiplied_matmul(x_likes, deg_ol, W_likes)

    agg_k0, agg_k1 = _scatter_kernel()(y_k0, y_k1,
                                       src_k.reshape(EROWS, CH),
                                       dst_k.reshape(EROWS, CH))
    agg_l0, agg_l1 = _scatter_kernel()(y_l0, y_l1,
                                       src_l.reshape(EROWS, CH),
                                       dst_l.reshape(EROWS, CH))

    return _epilogue(agg_k0, agg_k1, agg_l0, agg_l1, deg_ik, deg_il,
                     b_knows, b_likes, gamma, beta)
